# baseline (speedup 1.0000x reference)
import functools

import jax
import jax.numpy as jnp
from jax import lax
from jax.experimental import pallas as pl
from jax.experimental.pallas import tpu as pltpu
from jax.experimental.pallas import tpu_sc as plsc

N_N = 10000
N_PAD = 10240
E_PAD = 327680
CCH = 112
CNCH = 96
C_PAD = 2 * 16 * CNCH * CCH
IN_CH = 128
HID = 256
HALF = 128
NC = 2
NS = 16
ROWS_T = N_PAD // NS
ECH = 128
HCH = E_PAD // (NC * NS)
AGG_CH = E_PAD // NS // ECH

_MESH = plsc.VectorSubcoreMesh(
    core_axis_name="c", subcore_axis_name="s", num_cores=NC, num_subcores=NS)
_SC_PARAMS = pltpu.CompilerParams(needs_layout_passes=False)


def _hist_body(dst_hbm, cnt_hbm, dstv, histv):
    c = lax.axis_index("c")
    s = lax.axis_index("s")
    w = s * NC + c
    pltpu.sync_copy(dst_hbm.at[w], dstv)
    z16 = jnp.zeros((16,), jnp.float32)

    def zstep(i, carry):
        histv[pl.ds(i * 16, 16)] = z16
        return carry

    lax.fori_loop(0, N_PAD // 16, zstep, 0)
    ones16 = jnp.full((16,), 1.0, jnp.float32)

    def estep(i, carry):
        idx = dstv[pl.ds(i * 16, 16)]
        plsc.addupdate_scatter(histv, [idx], ones16)
        return carry

    lax.fori_loop(0, HCH // 16, estep, 0)
    pltpu.sync_copy(histv, cnt_hbm.at[w])


_hist = functools.partial(
    pl.kernel,
    out_type=jax.ShapeDtypeStruct((NC * NS, N_PAD), jnp.float32),
    mesh=_MESH,
    scratch_types=[
        pltpu.VMEM((HCH,), jnp.int32),
        pltpu.VMEM((N_PAD,), jnp.float32),
    ],
    compiler_params=_SC_PARAMS,
)(_hist_body)


GRP = 8
NGRP = AGG_CH // GRP
NMM = NGRP // 2


def _agg_body(gst_hbm, src_hbm, dst_hbm, out_hbm,
              srcg, dstg, rowb0, rowb1, acc_sh, gs0, gs1, ss0, ss1, is0, is1):
    c = lax.axis_index("c")
    s = lax.axis_index("s")
    rs_g = pl.ds(c * N_PAD + s * ROWS_T, ROWS_T)
    rs_a = pl.ds(s * ROWS_T, ROWS_T)
    base = c * (NS * AGG_CH) + s * AGG_CH
    dbase = s * AGG_CH
    rowbs = (rowb0, rowb1)
    gsems = (gs0, gs1)
    ssems = (ss0, ss1)
    isems = (is0, is1)

    def idx_start(h, grp_start):
        pltpu.async_copy(src_hbm.at[pl.ds(base + grp_start, GRP)],
                         srcg.at[h], isems[h])
        pltpu.async_copy(dst_hbm.at[pl.ds(dbase + grp_start, GRP)],
                         dstg.at[h], isems[h])

    def idx_wait(h, grp_start):
        pltpu.make_async_copy(src_hbm.at[pl.ds(base + grp_start, GRP)],
                              srcg.at[h], isems[h]).wait()
        pltpu.make_async_copy(dst_hbm.at[pl.ds(dbase + grp_start, GRP)],
                              dstg.at[h], isems[h]).wait()

    def gat_start(b, h, r):
        pltpu.async_copy(gst_hbm.at[srcg.at[h].at[r]], rowbs[b], gsems[b])

    def gat_wait(b, h, r):
        pltpu.make_async_copy(gst_hbm.at[srcg.at[h].at[r]],
                              rowbs[b], gsems[b]).wait()

    def sca_start(b, h, r):
        pltpu.async_copy(rowbs[b], acc_sh.at[dstg.at[h].at[r]],
                         ssems[b], add=True)

    def sca_wait(b, h, r):
        pltpu.make_async_copy(rowbs[b], acc_sh.at[dstg.at[h].at[r]],
                              ssems[b]).wait()

    pltpu.sync_copy(gst_hbm.at[rs_g], acc_sh.at[rs_a])
    idx_start(0, 0)
    plsc.subcore_barrier()

    def body(mm, carry):
        j0 = 16 * mm
        for k in range(16):
            h, r, b = k // 8, k % 8, k % 2
            if k < 2:
                @pl.when(mm > 0)
                def _(b=b, ph=1, pr=(k + 14) % 8):
                    sca_wait(b, ph, pr)
            else:
                sca_wait(b, (k - 2) // 8, (k - 2) % 8)
            if k == 0:
                idx_wait(0, j0)
            if k == 8:
                idx_wait(1, j0 + 8)
            gat_start(b, h, r)
            if k > 0:
                pb, ph, pr = (k - 1) % 2, (k - 1) // 8, (k - 1) % 8
                gat_wait(pb, ph, pr)
                sca_start(pb, ph, pr)
            if k == 2:
                idx_start(1, j0 + 8)
            if k == 9:
                @pl.when(mm < NMM - 1)
                def _():
                    idx_start(0, j0 + 16)
        gat_wait(1, 1, 7)
        sca_start(1, 1, 7)
        return carry

    lax.fori_loop(0, NMM, body, 0)
    sca_wait(0, 1, 6)
    sca_wait(1, 1, 7)
    plsc.subcore_barrier()
    pltpu.sync_copy(acc_sh.at[rs_a], out_hbm.at[rs_g])


_agg = functools.partial(
    pl.kernel,
    out_type=jax.ShapeDtypeStruct((2 * N_PAD, HALF), jnp.float32),
    mesh=_MESH,
    scratch_types=[
        pltpu.VMEM((2, GRP, ECH), jnp.int32),
        pltpu.VMEM((2, GRP, ECH), jnp.int32),
        pltpu.VMEM((ECH, HALF), jnp.float32),
        pltpu.VMEM((ECH, HALF), jnp.float32),
        pltpu.VMEM_SHARED((N_PAD, HALF), jnp.float32),
        pltpu.SemaphoreType.DMA,
        pltpu.SemaphoreType.DMA,
        pltpu.SemaphoreType.DMA,
        pltpu.SemaphoreType.DMA,
        pltpu.SemaphoreType.DMA,
        pltpu.SemaphoreType.DMA,
    ],
    compiler_params=_SC_PARAMS,
)(_agg_body)


def _cand_body(a_hbm, b_hbm, c0_hbm, c1_hbm, o_hbm,
               i0, i1, sb0, sb1, bb0, bb1,
               gsa0, gsa1, gsb0, gsb1, ws0, ws1, is0, is1):
    c = lax.axis_index("c")
    s = lax.axis_index("s")
    w = s * NC + c
    ibase = w * CNCH
    bufS = (sb0, sb1)
    bufB = (bb0, bb1)
    gsA = (gsa0, gsa1)
    gsB = (gsb0, gsb1)
    wss = (ws0, ws1)
    iss = (is0, is1)

    def idx_start(q, j):
        pltpu.async_copy(c0_hbm.at[ibase + j], i0.at[q], iss[q])
        pltpu.async_copy(c1_hbm.at[ibase + j], i1.at[q], iss[q])

    def idx_wait(q, j):
        pltpu.make_async_copy(c0_hbm.at[ibase + j], i0.at[q], iss[q]).wait()
        pltpu.make_async_copy(c1_hbm.at[ibase + j], i1.at[q], iss[q]).wait()

    def gath_start(b, q):
        pltpu.async_copy(a_hbm.at[i0.at[q]], bufS[b], gsA[b])
        pltpu.async_copy(b_hbm.at[i1.at[q]], bufB[b], gsB[b])

    def gath_wait(b, q):
        pltpu.make_async_copy(a_hbm.at[i0.at[q]], bufS[b], gsA[b]).wait()
        pltpu.make_async_copy(b_hbm.at[i1.at[q]], bufB[b], gsB[b]).wait()

    def wr_start(b, j):
        pltpu.async_copy(bufS[b], o_hbm.at[ibase + j], wss[b])

    def wr_wait(b, j):
        pltpu.make_async_copy(bufS[b], o_hbm.at[ibase + j], wss[b]).wait()

    def add_chunk(b):
        @plsc.parallel_loop(0, CCH, 1, unroll=4)
        def _(r):
            for k in range(HID // 16):
                sl = pl.ds(k * 16, 16)
                bufS[b][r, sl] = bufS[b][r, sl] + bufB[b][r, sl]

    def step(t, b, q, first_write_guard=None):
        gath_wait(1 - b, 1 - q)
        idx_start(1 - q, t + 1)
        if first_write_guard is None:
            wr_wait(b, t - 2)
        else:
            @pl.when(first_write_guard)
            def _():
                wr_wait(b, t - 2)
        idx_wait(q, t)
        gath_start(b, q)
        add_chunk(1 - b)
        wr_start(1 - b, t - 1)

    idx_start(0, 0)
    idx_wait(0, 0)
    gath_start(0, 0)
    idx_start(1, 1)

    def body(m, carry):
        step(2 * m + 1, 1, 1, first_write_guard=(m > 0))
        step(2 * m + 2, 0, 0)
        return carry

    lax.fori_loop(0, (CNCH - 2) // 2, body, 0)
    t = CNCH - 1
    gath_wait(0, 0)
    wr_wait(1, t - 2)
    idx_wait(1, t)
    gath_start(1, 1)
    add_chunk(0)
    wr_start(0, t - 1)
    gath_wait(1, 1)
    add_chunk(1)
    wr_start(1, t)
    wr_wait(0, t - 1)
    wr_wait(1, t)


_cand = functools.partial(
    pl.kernel,
    out_type=jax.ShapeDtypeStruct((NC * NS * CNCH, CCH, HID), jnp.float32),
    mesh=_MESH,
    scratch_types=[
        pltpu.VMEM((2, CCH), jnp.int32),
        pltpu.VMEM((2, CCH), jnp.int32),
        pltpu.VMEM((CCH, HID), jnp.float32),
        pltpu.VMEM((CCH, HID), jnp.float32),
        pltpu.VMEM((CCH, HID), jnp.float32),
        pltpu.VMEM((CCH, HID), jnp.float32),
        pltpu.SemaphoreType.DMA,
        pltpu.SemaphoreType.DMA,
        pltpu.SemaphoreType.DMA,
        pltpu.SemaphoreType.DMA,
        pltpu.SemaphoreType.DMA,
        pltpu.SemaphoreType.DMA,
        pltpu.SemaphoreType.DMA,
        pltpu.SemaphoreType.DMA,
    ],
    compiler_params=_SC_PARAMS,
)(_cand_body)


BLK = 1024


def _tc1_body(cnt_ref, x_ref, w_ref, g_ref, dinv_ref):
    deg = jnp.sum(cnt_ref[...], axis=0) + 1.0
    dinv = lax.rsqrt(deg)[:, None]
    dinv_ref[...] = dinv
    g = jnp.dot(x_ref[...], w_ref[...], preferred_element_type=jnp.float32) * dinv
    g_ref[0] = g[:, :HALF]
    g_ref[1] = g[:, HALF:]


def _tc1(cnt, x_p, W1):
    return pl.pallas_call(
        _tc1_body,
        grid=(N_PAD // BLK,),
        in_specs=[
            pl.BlockSpec((NC * NS, BLK), lambda i: (0, i)),
            pl.BlockSpec((BLK, IN_CH), lambda i: (i, 0)),
            pl.BlockSpec((IN_CH, HID), lambda i: (0, 0)),
        ],
        out_specs=[
            pl.BlockSpec((2, BLK, HALF), lambda i: (0, i, 0)),
            pl.BlockSpec((BLK, 1), lambda i: (i, 0)),
        ],
        out_shape=[
            jax.ShapeDtypeStruct((2, N_PAD, HALF), jnp.float32),
            jax.ShapeDtypeStruct((N_PAD, 1), jnp.float32),
        ],
    )(cnt, x_p, W1)


def _tc2_body(alo_ref, ahi_ref, dinv_ref, b_ref, w_ref, g_ref):
    a = jnp.concatenate([alo_ref[...], ahi_ref[...]], axis=1)
    dv = dinv_ref[...]
    h = jnp.maximum(a * dv + b_ref[...], 0.0)
    g = jnp.dot(h, w_ref[...], preferred_element_type=jnp.float32) * dv
    g_ref[0] = g[:, :HALF]
    g_ref[1] = g[:, HALF:]


NB = N_PAD // BLK


def _tc2(agg, dinv, b1, W2):
    return pl.pallas_call(
        _tc2_body,
        grid=(NB,),
        in_specs=[
            pl.BlockSpec((BLK, HALF), lambda i: (i, 0)),
            pl.BlockSpec((BLK, HALF), lambda i: (i + NB, 0)),
            pl.BlockSpec((BLK, 1), lambda i: (i, 0)),
            pl.BlockSpec((1, HID), lambda i: (0, 0)),
            pl.BlockSpec((HID, HID), lambda i: (0, 0)),
        ],
        out_specs=pl.BlockSpec((2, BLK, HALF), lambda i: (0, i, 0)),
        out_shape=jax.ShapeDtypeStruct((2, N_PAD, HALF), jnp.float32),
    )(agg, agg, dinv, b1, W2)


def _tc3_body(alo_ref, ahi_ref, dinv_ref, b_ref, wm1_ref, bm1_ref,
              h_ref, a_ref, bb_ref):
    a = jnp.concatenate([alo_ref[...], ahi_ref[...]], axis=1)
    dv = dinv_ref[...]
    h = jnp.maximum(a * dv + b_ref[...], 0.0)
    h_ref[...] = h
    a_ref[...] = jnp.dot(h, wm1_ref[:HID, :],
                         preferred_element_type=jnp.float32) + bm1_ref[...]
    bb_ref[...] = jnp.dot(h, wm1_ref[HID:, :],
                          preferred_element_type=jnp.float32)


def _tc3(agg, dinv, b2, Wm1, bm1):
    return pl.pallas_call(
        _tc3_body,
        grid=(NB,),
        in_specs=[
            pl.BlockSpec((BLK, HALF), lambda i: (i, 0)),
            pl.BlockSpec((BLK, HALF), lambda i: (i + NB, 0)),
            pl.BlockSpec((BLK, 1), lambda i: (i, 0)),
            pl.BlockSpec((1, HID), lambda i: (0, 0)),
            pl.BlockSpec((2 * HID, HID), lambda i: (0, 0)),
            pl.BlockSpec((1, HID), lambda i: (0, 0)),
        ],
        out_specs=[
            pl.BlockSpec((BLK, HID), lambda i: (i, 0)),
            pl.BlockSpec((BLK, HID), lambda i: (i, 0)),
            pl.BlockSpec((BLK, HID), lambda i: (i, 0)),
        ],
        out_shape=[
            jax.ShapeDtypeStruct((N_PAD, HID), jnp.float32),
            jax.ShapeDtypeStruct((N_PAD, HID), jnp.float32),
            jax.ShapeDtypeStruct((N_PAD, HID), jnp.float32),
        ],
    )(agg, agg, dinv, b2, Wm1, bm1)


CBLK = 4096


def _tc4_body(s_ref, wm2_ref, bm2_ref, o_ref):
    hid = jnp.maximum(s_ref[...], 0.0)
    o_ref[...] = jnp.dot(hid, wm2_ref[...],
                         preferred_element_type=jnp.float32) + bm2_ref[...]


def _tc4(ssum, Wm2, bm2):
    n_cls = Wm2.shape[1]
    return pl.pallas_call(
        _tc4_body,
        grid=(C_PAD // CBLK,),
        in_specs=[
            pl.BlockSpec((CBLK, HID), lambda i: (i, 0)),
            pl.BlockSpec((HID, n_cls), lambda i: (0, 0)),
            pl.BlockSpec((1, n_cls), lambda i: (0, 0)),
        ],
        out_specs=pl.BlockSpec((CBLK, n_cls), lambda i: (i, 0)),
        out_shape=jax.ShapeDtypeStruct((C_PAD, n_cls), jnp.float32),
    )(ssum, Wm2, bm2)


def kernel(x, edge_index, override_candidates, W1, b1, W2, b2, Wm1, bm1, Wm2, bm2):
    n_edges = edge_index.shape[1]
    n_cand = override_candidates.shape[0]

    src = edge_index[0]
    dst = edge_index[1]
    pe = E_PAD - n_edges
    src_p = jnp.concatenate([src, jnp.zeros((pe,), jnp.int32)])
    dst_p = jnp.concatenate([dst, jnp.full((pe,), N_N, jnp.int32)])
    dst_hist = dst_p.reshape(NC * NS, HCH)
    sbase = src_p.reshape(NS * AGG_CH, ECH)
    src_agg = jnp.concatenate([sbase, sbase + N_PAD], axis=0)
    dst_agg = dst_p.reshape(NS * AGG_CH, ECH)

    pc = C_PAD - n_cand
    zc = jnp.zeros((pc,), jnp.int32)
    c0_p = jnp.concatenate([override_candidates[:, 0], zc]).reshape(NC * NS * CNCH, CCH)
    c1_p = jnp.concatenate([override_candidates[:, 1], zc]).reshape(NC * NS * CNCH, CCH)

    x_p = jnp.concatenate(
        [x, jnp.zeros((N_PAD - x.shape[0], x.shape[1]), jnp.float32)])

    cnt = _hist(dst_hist)
    g1, dinv = _tc1(cnt, x_p, W1)
    a1 = _agg(g1.reshape(2 * N_PAD, HALF), src_agg, dst_agg)
    g2 = _tc2(a1, dinv, b1.reshape(1, -1), W2)
    a2 = _agg(g2.reshape(2 * N_PAD, HALF), src_agg, dst_agg)
    h2, A, B = _tc3(a2, dinv, b2.reshape(1, -1), Wm1, bm1.reshape(1, -1))
    ssum = _cand(A, B, c0_p, c1_p).reshape(C_PAD, HID)
    logits_p = _tc4(ssum, Wm2, bm2.reshape(1, -1))

    return (logits_p[:n_cand], override_candidates, h2[:N_N])


# speedup vs baseline: 1.0810x; 1.0810x over previous
import functools

import jax
import jax.numpy as jnp
from jax import lax
from jax.experimental import pallas as pl
from jax.experimental.pallas import tpu as pltpu
from jax.experimental.pallas import tpu_sc as plsc

N_N = 10000
N_PAD = 10240
E_PAD = 327680
CCH = 120
CNCH = 86
C_PAD = 2 * 16 * CNCH * CCH
IN_CH = 128
HID = 256
HALF = 128
NC = 2
NS = 16
ROWS_T = N_PAD // NS
ECH = 128
HCH = E_PAD // (NC * NS)
AGG_CH = E_PAD // NS // ECH

_MESH = plsc.VectorSubcoreMesh(
    core_axis_name="c", subcore_axis_name="s", num_cores=NC, num_subcores=NS)
_SC_PARAMS = pltpu.CompilerParams(needs_layout_passes=False)


def _hist_body(dst_hbm, cnt_hbm, dstv, histv):
    c = lax.axis_index("c")
    s = lax.axis_index("s")
    w = s * NC + c
    pltpu.sync_copy(dst_hbm.at[w], dstv)
    z16 = jnp.zeros((16,), jnp.float32)

    def zstep(i, carry):
        histv[pl.ds(i * 16, 16)] = z16
        return carry

    lax.fori_loop(0, N_PAD // 16, zstep, 0)
    ones16 = jnp.full((16,), 1.0, jnp.float32)

    def estep(i, carry):
        idx = dstv[pl.ds(i * 16, 16)]
        plsc.addupdate_scatter(histv, [idx], ones16)
        return carry

    lax.fori_loop(0, HCH // 16, estep, 0)
    pltpu.sync_copy(histv, cnt_hbm.at[w])


_hist = functools.partial(
    pl.kernel,
    out_type=jax.ShapeDtypeStruct((NC * NS, N_PAD), jnp.float32),
    mesh=_MESH,
    scratch_types=[
        pltpu.VMEM((HCH,), jnp.int32),
        pltpu.VMEM((N_PAD,), jnp.float32),
    ],
    compiler_params=_SC_PARAMS,
)(_hist_body)


GRP = 8
NGRP = AGG_CH // GRP
NMM = NGRP // 2


def _agg_body(gst_hbm, src_hbm, dst_hbm, out_hbm,
              srcg, dstg, rowb0, rowb1, acc_sh, gs0, gs1, ss0, ss1, is0, is1):
    c = lax.axis_index("c")
    s = lax.axis_index("s")
    rs_g = pl.ds(c * N_PAD + s * ROWS_T, ROWS_T)
    rs_a = pl.ds(s * ROWS_T, ROWS_T)
    base = c * (NS * AGG_CH) + s * AGG_CH
    dbase = s * AGG_CH
    rowbs = (rowb0, rowb1)
    gsems = (gs0, gs1)
    ssems = (ss0, ss1)
    isems = (is0, is1)

    def idx_start(h, grp_start):
        pltpu.async_copy(src_hbm.at[pl.ds(base + grp_start, GRP)],
                         srcg.at[h], isems[h])
        pltpu.async_copy(dst_hbm.at[pl.ds(dbase + grp_start, GRP)],
                         dstg.at[h], isems[h])

    def idx_wait(h, grp_start):
        pltpu.make_async_copy(src_hbm.at[pl.ds(base + grp_start, GRP)],
                              srcg.at[h], isems[h]).wait()
        pltpu.make_async_copy(dst_hbm.at[pl.ds(dbase + grp_start, GRP)],
                              dstg.at[h], isems[h]).wait()

    def gat_start(b, h, r):
        pltpu.async_copy(gst_hbm.at[srcg.at[h].at[r]], rowbs[b], gsems[b])

    def gat_wait(b, h, r):
        pltpu.make_async_copy(gst_hbm.at[srcg.at[h].at[r]],
                              rowbs[b], gsems[b]).wait()

    def sca_start(b, h, r):
        pltpu.async_copy(rowbs[b], acc_sh.at[dstg.at[h].at[r]],
                         ssems[b], add=True)

    def sca_wait(b, h, r):
        pltpu.make_async_copy(rowbs[b], acc_sh.at[dstg.at[h].at[r]],
                              ssems[b]).wait()

    pltpu.sync_copy(gst_hbm.at[rs_g], acc_sh.at[rs_a])
    idx_start(0, 0)
    plsc.subcore_barrier()

    def body(mm, carry):
        j0 = 16 * mm
        for k in range(16):
            h, r, b = k // 8, k % 8, k % 2
            if k < 2:
                @pl.when(mm > 0)
                def _(b=b, ph=1, pr=(k + 14) % 8):
                    sca_wait(b, ph, pr)
            else:
                sca_wait(b, (k - 2) // 8, (k - 2) % 8)
            if k == 0:
                idx_wait(0, j0)
            if k == 8:
                idx_wait(1, j0 + 8)
            gat_start(b, h, r)
            if k > 0:
                pb, ph, pr = (k - 1) % 2, (k - 1) // 8, (k - 1) % 8
                gat_wait(pb, ph, pr)
                sca_start(pb, ph, pr)
            if k == 2:
                idx_start(1, j0 + 8)
            if k == 9:
                @pl.when(mm < NMM - 1)
                def _():
                    idx_start(0, j0 + 16)
        gat_wait(1, 1, 7)
        sca_start(1, 1, 7)
        return carry

    lax.fori_loop(0, NMM, body, 0)
    sca_wait(0, 1, 6)
    sca_wait(1, 1, 7)
    plsc.subcore_barrier()
    pltpu.sync_copy(acc_sh.at[rs_a], out_hbm.at[rs_g])


_agg = functools.partial(
    pl.kernel,
    out_type=jax.ShapeDtypeStruct((2 * N_PAD, HALF), jnp.float32),
    mesh=_MESH,
    scratch_types=[
        pltpu.VMEM((2, GRP, ECH), jnp.int32),
        pltpu.VMEM((2, GRP, ECH), jnp.int32),
        pltpu.VMEM((ECH, HALF), jnp.float32),
        pltpu.VMEM((ECH, HALF), jnp.float32),
        pltpu.VMEM_SHARED((N_PAD, HALF), jnp.float32),
        pltpu.SemaphoreType.DMA,
        pltpu.SemaphoreType.DMA,
        pltpu.SemaphoreType.DMA,
        pltpu.SemaphoreType.DMA,
        pltpu.SemaphoreType.DMA,
        pltpu.SemaphoreType.DMA,
    ],
    compiler_params=_SC_PARAMS,
)(_agg_body)


def _cand_body(a_hbm, b_hbm, c0_hbm, c1_hbm, oa_hbm, ob_hbm,
               i0, i1, ba0, ba1, bb0, bb1,
               gsa0, gsa1, gsb0, gsb1, wa0, wa1, wb0, wb1, is0, is1):
    c = lax.axis_index("c")
    s = lax.axis_index("s")
    w = s * NC + c
    ibase = w * CNCH
    bufA = (ba0, ba1)
    bufB = (bb0, bb1)
    gsA = (gsa0, gsa1)
    gsB = (gsb0, gsb1)
    wsA = (wa0, wa1)
    wsB = (wb0, wb1)
    iss = (is0, is1)

    def idx_start(q, j):
        pltpu.async_copy(c0_hbm.at[ibase + j], i0.at[q], iss[q])
        pltpu.async_copy(c1_hbm.at[ibase + j], i1.at[q], iss[q])

    def idx_wait(q, j):
        pltpu.make_async_copy(c0_hbm.at[ibase + j], i0.at[q], iss[q]).wait()
        pltpu.make_async_copy(c1_hbm.at[ibase + j], i1.at[q], iss[q]).wait()

    def gath_start(b, q):
        pltpu.async_copy(a_hbm.at[i0.at[q]], bufA[b], gsA[b])
        pltpu.async_copy(b_hbm.at[i1.at[q]], bufB[b], gsB[b])

    def gath_wait(b, q):
        pltpu.make_async_copy(a_hbm.at[i0.at[q]], bufA[b], gsA[b]).wait()
        pltpu.make_async_copy(b_hbm.at[i1.at[q]], bufB[b], gsB[b]).wait()

    def wr_start(b, j):
        pltpu.async_copy(bufA[b], oa_hbm.at[ibase + j], wsA[b])
        pltpu.async_copy(bufB[b], ob_hbm.at[ibase + j], wsB[b])

    def wr_wait(b, j):
        pltpu.make_async_copy(bufA[b], oa_hbm.at[ibase + j], wsA[b]).wait()
        pltpu.make_async_copy(bufB[b], ob_hbm.at[ibase + j], wsB[b]).wait()

    def step(t, b, q, first_write_guard=None):
        gath_wait(1 - b, 1 - q)
        idx_start(1 - q, t + 1)
        if first_write_guard is None:
            wr_wait(b, t - 2)
        else:
            @pl.when(first_write_guard)
            def _():
                wr_wait(b, t - 2)
        idx_wait(q, t)
        gath_start(b, q)
        wr_start(1 - b, t - 1)

    idx_start(0, 0)
    idx_wait(0, 0)
    gath_start(0, 0)
    idx_start(1, 1)

    def body(m, carry):
        step(2 * m + 1, 1, 1, first_write_guard=(m > 0))
        step(2 * m + 2, 0, 0)
        return carry

    lax.fori_loop(0, (CNCH - 2) // 2, body, 0)
    t = CNCH - 1
    gath_wait(0, 0)
    wr_wait(1, t - 2)
    idx_wait(1, t)
    gath_start(1, 1)
    wr_start(0, t - 1)
    gath_wait(1, 1)
    wr_start(1, t)
    wr_wait(0, t - 1)
    wr_wait(1, t)


_cand = functools.partial(
    pl.kernel,
    out_type=[jax.ShapeDtypeStruct((NC * NS * CNCH, CCH, HID), jnp.float32),
              jax.ShapeDtypeStruct((NC * NS * CNCH, CCH, HID), jnp.float32)],
    mesh=_MESH,
    scratch_types=[
        pltpu.VMEM((2, CCH), jnp.int32),
        pltpu.VMEM((2, CCH), jnp.int32),
        pltpu.VMEM((CCH, HID), jnp.float32),
        pltpu.VMEM((CCH, HID), jnp.float32),
        pltpu.VMEM((CCH, HID), jnp.float32),
        pltpu.VMEM((CCH, HID), jnp.float32),
        pltpu.SemaphoreType.DMA,
        pltpu.SemaphoreType.DMA,
        pltpu.SemaphoreType.DMA,
        pltpu.SemaphoreType.DMA,
        pltpu.SemaphoreType.DMA,
        pltpu.SemaphoreType.DMA,
        pltpu.SemaphoreType.DMA,
        pltpu.SemaphoreType.DMA,
        pltpu.SemaphoreType.DMA,
        pltpu.SemaphoreType.DMA,
    ],
    compiler_params=_SC_PARAMS,
)(_cand_body)


BLK = 1024


def _tc1_body(cnt_ref, x_ref, w_ref, g_ref, dinv_ref):
    deg = jnp.sum(cnt_ref[...], axis=0) + 1.0
    dinv = lax.rsqrt(deg)[:, None]
    dinv_ref[...] = dinv
    g = jnp.dot(x_ref[...], w_ref[...], preferred_element_type=jnp.float32) * dinv
    g_ref[0] = g[:, :HALF]
    g_ref[1] = g[:, HALF:]


def _tc1(cnt, x_p, W1):
    return pl.pallas_call(
        _tc1_body,
        grid=(N_PAD // BLK,),
        in_specs=[
            pl.BlockSpec((NC * NS, BLK), lambda i: (0, i)),
            pl.BlockSpec((BLK, IN_CH), lambda i: (i, 0)),
            pl.BlockSpec((IN_CH, HID), lambda i: (0, 0)),
        ],
        out_specs=[
            pl.BlockSpec((2, BLK, HALF), lambda i: (0, i, 0)),
            pl.BlockSpec((BLK, 1), lambda i: (i, 0)),
        ],
        out_shape=[
            jax.ShapeDtypeStruct((2, N_PAD, HALF), jnp.float32),
            jax.ShapeDtypeStruct((N_PAD, 1), jnp.float32),
        ],
    )(cnt, x_p, W1)


def _tc2_body(alo_ref, ahi_ref, dinv_ref, b_ref, w_ref, g_ref):
    a = jnp.concatenate([alo_ref[...], ahi_ref[...]], axis=1)
    dv = dinv_ref[...]
    h = jnp.maximum(a * dv + b_ref[...], 0.0)
    g = jnp.dot(h, w_ref[...], preferred_element_type=jnp.float32) * dv
    g_ref[0] = g[:, :HALF]
    g_ref[1] = g[:, HALF:]


NB = N_PAD // BLK


def _tc2(agg, dinv, b1, W2):
    return pl.pallas_call(
        _tc2_body,
        grid=(NB,),
        in_specs=[
            pl.BlockSpec((BLK, HALF), lambda i: (i, 0)),
            pl.BlockSpec((BLK, HALF), lambda i: (i + NB, 0)),
            pl.BlockSpec((BLK, 1), lambda i: (i, 0)),
            pl.BlockSpec((1, HID), lambda i: (0, 0)),
            pl.BlockSpec((HID, HID), lambda i: (0, 0)),
        ],
        out_specs=pl.BlockSpec((2, BLK, HALF), lambda i: (0, i, 0)),
        out_shape=jax.ShapeDtypeStruct((2, N_PAD, HALF), jnp.float32),
    )(agg, agg, dinv, b1, W2)


def _tc3_body(alo_ref, ahi_ref, dinv_ref, b_ref, wm1_ref, bm1_ref,
              h_ref, a_ref, bb_ref):
    a = jnp.concatenate([alo_ref[...], ahi_ref[...]], axis=1)
    dv = dinv_ref[...]
    h = jnp.maximum(a * dv + b_ref[...], 0.0)
    h_ref[...] = h
    a_ref[...] = jnp.dot(h, wm1_ref[:HID, :],
                         preferred_element_type=jnp.float32) + bm1_ref[...]
    bb_ref[...] = jnp.dot(h, wm1_ref[HID:, :],
                          preferred_element_type=jnp.float32)


def _tc3(agg, dinv, b2, Wm1, bm1):
    return pl.pallas_call(
        _tc3_body,
        grid=(NB,),
        in_specs=[
            pl.BlockSpec((BLK, HALF), lambda i: (i, 0)),
            pl.BlockSpec((BLK, HALF), lambda i: (i + NB, 0)),
            pl.BlockSpec((BLK, 1), lambda i: (i, 0)),
            pl.BlockSpec((1, HID), lambda i: (0, 0)),
            pl.BlockSpec((2 * HID, HID), lambda i: (0, 0)),
            pl.BlockSpec((1, HID), lambda i: (0, 0)),
        ],
        out_specs=[
            pl.BlockSpec((BLK, HID), lambda i: (i, 0)),
            pl.BlockSpec((BLK, HID), lambda i: (i, 0)),
            pl.BlockSpec((BLK, HID), lambda i: (i, 0)),
        ],
        out_shape=[
            jax.ShapeDtypeStruct((N_PAD, HID), jnp.float32),
            jax.ShapeDtypeStruct((N_PAD, HID), jnp.float32),
            jax.ShapeDtypeStruct((N_PAD, HID), jnp.float32),
        ],
    )(agg, agg, dinv, b2, Wm1, bm1)


CBLK = 1920


def _tc4_body(sa_ref, sb_ref, wm2_ref, bm2_ref, o_ref):
    hid = jnp.maximum(sa_ref[...] + sb_ref[...], 0.0)
    o_ref[...] = jnp.dot(hid, wm2_ref[...],
                         preferred_element_type=jnp.float32) + bm2_ref[...]


def _tc4(sa, sb, Wm2, bm2):
    n_cls = Wm2.shape[1]
    return pl.pallas_call(
        _tc4_body,
        grid=(C_PAD // CBLK,),
        in_specs=[
            pl.BlockSpec((CBLK, HID), lambda i: (i, 0)),
            pl.BlockSpec((CBLK, HID), lambda i: (i, 0)),
            pl.BlockSpec((HID, n_cls), lambda i: (0, 0)),
            pl.BlockSpec((1, n_cls), lambda i: (0, 0)),
        ],
        out_specs=pl.BlockSpec((CBLK, n_cls), lambda i: (i, 0)),
        out_shape=jax.ShapeDtypeStruct((C_PAD, n_cls), jnp.float32),
    )(sa, sb, Wm2, bm2)


def kernel(x, edge_index, override_candidates, W1, b1, W2, b2, Wm1, bm1, Wm2, bm2):
    n_edges = edge_index.shape[1]
    n_cand = override_candidates.shape[0]

    src = edge_index[0]
    dst = edge_index[1]
    pe = E_PAD - n_edges
    src_p = jnp.concatenate([src, jnp.zeros((pe,), jnp.int32)])
    dst_p = jnp.concatenate([dst, jnp.full((pe,), N_N, jnp.int32)])
    dst_hist = dst_p.reshape(NC * NS, HCH)
    sbase = src_p.reshape(NS * AGG_CH, ECH)
    src_agg = jnp.concatenate([sbase, sbase + N_PAD], axis=0)
    dst_agg = dst_p.reshape(NS * AGG_CH, ECH)

    pc = C_PAD - n_cand
    zc = jnp.zeros((pc,), jnp.int32)
    c0_p = jnp.concatenate([override_candidates[:, 0], zc]).reshape(NC * NS * CNCH, CCH)
    c1_p = jnp.concatenate([override_candidates[:, 1], zc]).reshape(NC * NS * CNCH, CCH)

    x_p = jnp.concatenate(
        [x, jnp.zeros((N_PAD - x.shape[0], x.shape[1]), jnp.float32)])

    cnt = _hist(dst_hist)
    g1, dinv = _tc1(cnt, x_p, W1)
    a1 = _agg(g1.reshape(2 * N_PAD, HALF), src_agg, dst_agg)
    g2 = _tc2(a1, dinv, b1.reshape(1, -1), W2)
    a2 = _agg(g2.reshape(2 * N_PAD, HALF), src_agg, dst_agg)
    h2, A, B = _tc3(a2, dinv, b2.reshape(1, -1), Wm1, bm1.reshape(1, -1))
    sa, sb = _cand(A, B, c0_p, c1_p)
    logits_p = _tc4(sa.reshape(C_PAD, HID), sb.reshape(C_PAD, HID),
                    Wm2, bm2.reshape(1, -1))

    return (logits_p[:n_cand], override_candidates, h2[:N_N])


# speedup vs baseline: 1.1958x; 1.1062x over previous
import functools

import jax
import jax.numpy as jnp
from jax import lax
from jax.experimental import pallas as pl
from jax.experimental.pallas import tpu as pltpu
from jax.experimental.pallas import tpu_sc as plsc

N_N = 10000
N_PAD = 10240
E_PAD = 327680
CCH = 120
CNCH = 86
C_PAD = 2 * 16 * CNCH * CCH
IN_CH = 128
HID = 256
HALF = 128
NC = 2
NS = 16
ROWS_T = N_PAD // NS
ECH = 128
HCH = E_PAD // (NC * NS)
AGG_CH = E_PAD // NS // ECH

_MESH = plsc.VectorSubcoreMesh(
    core_axis_name="c", subcore_axis_name="s", num_cores=NC, num_subcores=NS)
_SC_PARAMS = pltpu.CompilerParams(needs_layout_passes=False)


def _hist_body(dst_hbm, cnt_hbm, dstv, histv):
    c = lax.axis_index("c")
    s = lax.axis_index("s")
    w = s * NC + c
    pltpu.sync_copy(dst_hbm.at[w], dstv)
    z16 = jnp.zeros((16,), jnp.float32)

    def zstep(i, carry):
        histv[pl.ds(i * 16, 16)] = z16
        return carry

    lax.fori_loop(0, N_PAD // 16, zstep, 0)
    ones16 = jnp.full((16,), 1.0, jnp.float32)

    def estep(i, carry):
        idx = dstv[pl.ds(i * 16, 16)]
        plsc.addupdate_scatter(histv, [idx], ones16)
        return carry

    lax.fori_loop(0, HCH // 16, estep, 0)
    pltpu.sync_copy(histv, cnt_hbm.at[w])


_hist = functools.partial(
    pl.kernel,
    out_type=jax.ShapeDtypeStruct((NC * NS, N_PAD), jnp.float32),
    mesh=_MESH,
    scratch_types=[
        pltpu.VMEM((HCH,), jnp.int32),
        pltpu.VMEM((N_PAD,), jnp.float32),
    ],
    compiler_params=_SC_PARAMS,
)(_hist_body)


GRP = 8
NGRP = AGG_CH // GRP
NMM = NGRP // 2


def _agg_body(gst_hbm, src_hbm, dst_hbm, out_hbm,
              srcg, dstg, rowb0, rowb1, acc_sh, gs0, gs1, ss0, ss1, is0, is1):
    c = lax.axis_index("c")
    s = lax.axis_index("s")
    rs_g = pl.ds(c * N_PAD + s * ROWS_T, ROWS_T)
    rs_a = pl.ds(s * ROWS_T, ROWS_T)
    base = c * (NS * AGG_CH) + s * AGG_CH
    dbase = s * AGG_CH
    rowbs = (rowb0, rowb1)
    gsems = (gs0, gs1)
    ssems = (ss0, ss1)
    isems = (is0, is1)

    def idx_start(h, grp_start):
        pltpu.async_copy(src_hbm.at[pl.ds(base + grp_start, GRP)],
                         srcg.at[h], isems[h])
        pltpu.async_copy(dst_hbm.at[pl.ds(dbase + grp_start, GRP)],
                         dstg.at[h], isems[h])

    def idx_wait(h, grp_start):
        pltpu.make_async_copy(src_hbm.at[pl.ds(base + grp_start, GRP)],
                              srcg.at[h], isems[h]).wait()
        pltpu.make_async_copy(dst_hbm.at[pl.ds(dbase + grp_start, GRP)],
                              dstg.at[h], isems[h]).wait()

    def gat_start(b, h, r):
        pltpu.async_copy(gst_hbm.at[srcg.at[h].at[r]], rowbs[b], gsems[b])

    def gat_wait(b, h, r):
        pltpu.make_async_copy(gst_hbm.at[srcg.at[h].at[r]],
                              rowbs[b], gsems[b]).wait()

    def sca_start(b, h, r):
        pltpu.async_copy(rowbs[b], acc_sh.at[dstg.at[h].at[r]],
                         ssems[b], add=True)

    def sca_wait(b, h, r):
        pltpu.make_async_copy(rowbs[b], acc_sh.at[dstg.at[h].at[r]],
                              ssems[b]).wait()

    pltpu.sync_copy(gst_hbm.at[rs_g], acc_sh.at[rs_a])
    idx_start(0, 0)
    plsc.subcore_barrier()

    def body(mm, carry):
        j0 = 16 * mm
        for k in range(16):
            h, r, b = k // 8, k % 8, k % 2
            if k < 2:
                @pl.when(mm > 0)
                def _(b=b, ph=1, pr=(k + 14) % 8):
                    sca_wait(b, ph, pr)
            else:
                sca_wait(b, (k - 2) // 8, (k - 2) % 8)
            if k == 0:
                idx_wait(0, j0)
            if k == 8:
                idx_wait(1, j0 + 8)
            gat_start(b, h, r)
            if k > 0:
                pb, ph, pr = (k - 1) % 2, (k - 1) // 8, (k - 1) % 8
                gat_wait(pb, ph, pr)
                sca_start(pb, ph, pr)
            if k == 2:
                idx_start(1, j0 + 8)
            if k == 9:
                @pl.when(mm < NMM - 1)
                def _():
                    idx_start(0, j0 + 16)
        gat_wait(1, 1, 7)
        sca_start(1, 1, 7)
        return carry

    lax.fori_loop(0, NMM, body, 0)
    sca_wait(0, 1, 6)
    sca_wait(1, 1, 7)
    plsc.subcore_barrier()
    pltpu.sync_copy(acc_sh.at[rs_a], out_hbm.at[rs_g])


_agg = functools.partial(
    pl.kernel,
    out_type=jax.ShapeDtypeStruct((2 * N_PAD, HALF), jnp.float32),
    mesh=_MESH,
    scratch_types=[
        pltpu.VMEM((2, GRP, ECH), jnp.int32),
        pltpu.VMEM((2, GRP, ECH), jnp.int32),
        pltpu.VMEM((ECH, HALF), jnp.float32),
        pltpu.VMEM((ECH, HALF), jnp.float32),
        pltpu.VMEM_SHARED((N_PAD, HALF), jnp.float32),
        pltpu.SemaphoreType.DMA,
        pltpu.SemaphoreType.DMA,
        pltpu.SemaphoreType.DMA,
        pltpu.SemaphoreType.DMA,
        pltpu.SemaphoreType.DMA,
        pltpu.SemaphoreType.DMA,
    ],
    compiler_params=_SC_PARAMS,
)(_agg_body)


def _cand_body(a_hbm, b_hbm, c0_hbm, c1_hbm, oa_hbm, ob_hbm,
               i0, i1, ba0, ba1, bb0, bb1,
               gsa0, gsa1, gsb0, gsb1, wa0, wa1, wb0, wb1, is0, is1):
    c = lax.axis_index("c")
    s = lax.axis_index("s")
    w = s * NC + c
    ibase = w * CNCH
    bufA = (ba0, ba1)
    bufB = (bb0, bb1)
    gsA = (gsa0, gsa1)
    gsB = (gsb0, gsb1)
    wsA = (wa0, wa1)
    wsB = (wb0, wb1)
    iss = (is0, is1)

    def idx_start(q, j):
        pltpu.async_copy(c0_hbm.at[ibase + j], i0.at[q], iss[q])
        pltpu.async_copy(c1_hbm.at[ibase + j], i1.at[q], iss[q])

    def idx_wait(q, j):
        pltpu.make_async_copy(c0_hbm.at[ibase + j], i0.at[q], iss[q]).wait()
        pltpu.make_async_copy(c1_hbm.at[ibase + j], i1.at[q], iss[q]).wait()

    def gath_start(b, q):
        pltpu.async_copy(a_hbm.at[i0.at[q]], bufA[b], gsA[b])
        pltpu.async_copy(b_hbm.at[i1.at[q]], bufB[b], gsB[b])

    def gath_wait(b, q):
        pltpu.make_async_copy(a_hbm.at[i0.at[q]], bufA[b], gsA[b]).wait()
        pltpu.make_async_copy(b_hbm.at[i1.at[q]], bufB[b], gsB[b]).wait()

    def wr_start(b, j):
        pltpu.async_copy(bufA[b], oa_hbm.at[ibase + j], wsA[b])
        pltpu.async_copy(bufB[b], ob_hbm.at[ibase + j], wsB[b])

    def wr_wait(b, j):
        pltpu.make_async_copy(bufA[b], oa_hbm.at[ibase + j], wsA[b]).wait()
        pltpu.make_async_copy(bufB[b], ob_hbm.at[ibase + j], wsB[b]).wait()

    def step(t, b, q, first_write_guard=None):
        gath_wait(1 - b, 1 - q)
        idx_start(1 - q, t + 1)
        if first_write_guard is None:
            wr_wait(b, t - 2)
        else:
            @pl.when(first_write_guard)
            def _():
                wr_wait(b, t - 2)
        idx_wait(q, t)
        gath_start(b, q)
        wr_start(1 - b, t - 1)

    idx_start(0, 0)
    idx_wait(0, 0)
    gath_start(0, 0)
    idx_start(1, 1)

    def body(m, carry):
        step(2 * m + 1, 1, 1, first_write_guard=(m > 0))
        step(2 * m + 2, 0, 0)
        return carry

    lax.fori_loop(0, (CNCH - 2) // 2, body, 0)
    t = CNCH - 1
    gath_wait(0, 0)
    wr_wait(1, t - 2)
    idx_wait(1, t)
    gath_start(1, 1)
    wr_start(0, t - 1)
    gath_wait(1, 1)
    wr_start(1, t)
    wr_wait(0, t - 1)
    wr_wait(1, t)


_cand = functools.partial(
    pl.kernel,
    out_type=[jax.ShapeDtypeStruct((NC * NS * CNCH, CCH, HALF), jnp.uint32),
              jax.ShapeDtypeStruct((NC * NS * CNCH, CCH, HALF), jnp.uint32)],
    mesh=_MESH,
    scratch_types=[
        pltpu.VMEM((2, CCH), jnp.int32),
        pltpu.VMEM((2, CCH), jnp.int32),
        pltpu.VMEM((CCH, HALF), jnp.uint32),
        pltpu.VMEM((CCH, HALF), jnp.uint32),
        pltpu.VMEM((CCH, HALF), jnp.uint32),
        pltpu.VMEM((CCH, HALF), jnp.uint32),
        pltpu.SemaphoreType.DMA,
        pltpu.SemaphoreType.DMA,
        pltpu.SemaphoreType.DMA,
        pltpu.SemaphoreType.DMA,
        pltpu.SemaphoreType.DMA,
        pltpu.SemaphoreType.DMA,
        pltpu.SemaphoreType.DMA,
        pltpu.SemaphoreType.DMA,
        pltpu.SemaphoreType.DMA,
        pltpu.SemaphoreType.DMA,
    ],
    compiler_params=_SC_PARAMS,
)(_cand_body)


BLK = 1024


def _tc1_body(cnt_ref, x_ref, w_ref, g_ref, dinv_ref):
    deg = jnp.sum(cnt_ref[...], axis=0) + 1.0
    dinv = lax.rsqrt(deg)[:, None]
    dinv_ref[...] = dinv
    g = jnp.dot(x_ref[...], w_ref[...], preferred_element_type=jnp.float32) * dinv
    g_ref[0] = g[:, :HALF]
    g_ref[1] = g[:, HALF:]


def _tc1(cnt, x_p, W1):
    return pl.pallas_call(
        _tc1_body,
        grid=(N_PAD // BLK,),
        in_specs=[
            pl.BlockSpec((NC * NS, BLK), lambda i: (0, i)),
            pl.BlockSpec((BLK, IN_CH), lambda i: (i, 0)),
            pl.BlockSpec((IN_CH, HID), lambda i: (0, 0)),
        ],
        out_specs=[
            pl.BlockSpec((2, BLK, HALF), lambda i: (0, i, 0)),
            pl.BlockSpec((BLK, 1), lambda i: (i, 0)),
        ],
        out_shape=[
            jax.ShapeDtypeStruct((2, N_PAD, HALF), jnp.float32),
            jax.ShapeDtypeStruct((N_PAD, 1), jnp.float32),
        ],
    )(cnt, x_p, W1)


def _tc2_body(alo_ref, ahi_ref, dinv_ref, b_ref, w_ref, g_ref):
    a = jnp.concatenate([alo_ref[...], ahi_ref[...]], axis=1)
    dv = dinv_ref[...]
    h = jnp.maximum(a * dv + b_ref[...], 0.0)
    g = jnp.dot(h, w_ref[...], preferred_element_type=jnp.float32) * dv
    g_ref[0] = g[:, :HALF]
    g_ref[1] = g[:, HALF:]


NB = N_PAD // BLK


def _tc2(agg, dinv, b1, W2):
    return pl.pallas_call(
        _tc2_body,
        grid=(NB,),
        in_specs=[
            pl.BlockSpec((BLK, HALF), lambda i: (i, 0)),
            pl.BlockSpec((BLK, HALF), lambda i: (i + NB, 0)),
            pl.BlockSpec((BLK, 1), lambda i: (i, 0)),
            pl.BlockSpec((1, HID), lambda i: (0, 0)),
            pl.BlockSpec((HID, HID), lambda i: (0, 0)),
        ],
        out_specs=pl.BlockSpec((2, BLK, HALF), lambda i: (0, i, 0)),
        out_shape=jax.ShapeDtypeStruct((2, N_PAD, HALF), jnp.float32),
    )(agg, agg, dinv, b1, W2)


def _pack_bf16(x):
    xb = x.astype(jnp.bfloat16)
    lo = lax.bitcast_convert_type(xb[:, :HALF], jnp.uint16).astype(jnp.uint32)
    hi = lax.bitcast_convert_type(xb[:, HALF:], jnp.uint16).astype(jnp.uint32)
    return lo | (hi << 16)


def _unpack_bf16(p):
    lo = lax.bitcast_convert_type((p & 0xFFFF).astype(jnp.uint16),
                                  jnp.bfloat16).astype(jnp.float32)
    hi = lax.bitcast_convert_type((p >> 16).astype(jnp.uint16),
                                  jnp.bfloat16).astype(jnp.float32)
    return lo, hi


def _tc3_body(alo_ref, ahi_ref, dinv_ref, b_ref, wm1_ref, bm1_ref,
              h_ref, a_ref, bb_ref):
    a = jnp.concatenate([alo_ref[...], ahi_ref[...]], axis=1)
    dv = dinv_ref[...]
    h = jnp.maximum(a * dv + b_ref[...], 0.0)
    h_ref[...] = h
    a_ref[...] = _pack_bf16(jnp.dot(h, wm1_ref[:HID, :],
                                    preferred_element_type=jnp.float32)
                            + bm1_ref[...])
    bb_ref[...] = _pack_bf16(jnp.dot(h, wm1_ref[HID:, :],
                                     preferred_element_type=jnp.float32))


def _tc3(agg, dinv, b2, Wm1, bm1):
    return pl.pallas_call(
        _tc3_body,
        grid=(NB,),
        in_specs=[
            pl.BlockSpec((BLK, HALF), lambda i: (i, 0)),
            pl.BlockSpec((BLK, HALF), lambda i: (i + NB, 0)),
            pl.BlockSpec((BLK, 1), lambda i: (i, 0)),
            pl.BlockSpec((1, HID), lambda i: (0, 0)),
            pl.BlockSpec((2 * HID, HID), lambda i: (0, 0)),
            pl.BlockSpec((1, HID), lambda i: (0, 0)),
        ],
        out_specs=[
            pl.BlockSpec((BLK, HID), lambda i: (i, 0)),
            pl.BlockSpec((BLK, HALF), lambda i: (i, 0)),
            pl.BlockSpec((BLK, HALF), lambda i: (i, 0)),
        ],
        out_shape=[
            jax.ShapeDtypeStruct((N_PAD, HID), jnp.float32),
            jax.ShapeDtypeStruct((N_PAD, HALF), jnp.uint32),
            jax.ShapeDtypeStruct((N_PAD, HALF), jnp.uint32),
        ],
    )(agg, agg, dinv, b2, Wm1, bm1)


CBLK = 1920


def _tc4_body(sa_ref, sb_ref, wm2_ref, bm2_ref, o_ref):
    la, ha = _unpack_bf16(sa_ref[...])
    lb, hb = _unpack_bf16(sb_ref[...])
    hid = jnp.maximum(jnp.concatenate([la + lb, ha + hb], axis=1), 0.0)
    o_ref[...] = jnp.dot(hid, wm2_ref[...],
                         preferred_element_type=jnp.float32) + bm2_ref[...]


def _tc4(sa, sb, Wm2, bm2):
    n_cls = Wm2.shape[1]
    return pl.pallas_call(
        _tc4_body,
        grid=(C_PAD // CBLK,),
        in_specs=[
            pl.BlockSpec((CBLK, HALF), lambda i: (i, 0)),
            pl.BlockSpec((CBLK, HALF), lambda i: (i, 0)),
            pl.BlockSpec((HID, n_cls), lambda i: (0, 0)),
            pl.BlockSpec((1, n_cls), lambda i: (0, 0)),
        ],
        out_specs=pl.BlockSpec((CBLK, n_cls), lambda i: (i, 0)),
        out_shape=jax.ShapeDtypeStruct((C_PAD, n_cls), jnp.float32),
    )(sa, sb, Wm2, bm2)


def kernel(x, edge_index, override_candidates, W1, b1, W2, b2, Wm1, bm1, Wm2, bm2):
    n_edges = edge_index.shape[1]
    n_cand = override_candidates.shape[0]

    src = edge_index[0]
    dst = edge_index[1]
    pe = E_PAD - n_edges
    src_p = jnp.concatenate([src, jnp.zeros((pe,), jnp.int32)])
    dst_p = jnp.concatenate([dst, jnp.full((pe,), N_N, jnp.int32)])
    dst_hist = dst_p.reshape(NC * NS, HCH)
    sbase = src_p.reshape(NS * AGG_CH, ECH)
    src_agg = jnp.concatenate([sbase, sbase + N_PAD], axis=0)
    dst_agg = dst_p.reshape(NS * AGG_CH, ECH)

    pc = C_PAD - n_cand
    zc = jnp.zeros((pc,), jnp.int32)
    c0_p = jnp.concatenate([override_candidates[:, 0], zc]).reshape(NC * NS * CNCH, CCH)
    c1_p = jnp.concatenate([override_candidates[:, 1], zc]).reshape(NC * NS * CNCH, CCH)

    x_p = jnp.concatenate(
        [x, jnp.zeros((N_PAD - x.shape[0], x.shape[1]), jnp.float32)])

    cnt = _hist(dst_hist)
    g1, dinv = _tc1(cnt, x_p, W1)
    a1 = _agg(g1.reshape(2 * N_PAD, HALF), src_agg, dst_agg)
    g2 = _tc2(a1, dinv, b1.reshape(1, -1), W2)
    a2 = _agg(g2.reshape(2 * N_PAD, HALF), src_agg, dst_agg)
    h2, A, B = _tc3(a2, dinv, b2.reshape(1, -1), Wm1, bm1.reshape(1, -1))
    sa, sb = _cand(A, B, c0_p, c1_p)
    logits_p = _tc4(sa.reshape(C_PAD, HALF), sb.reshape(C_PAD, HALF),
                    Wm2, bm2.reshape(1, -1))

    return (logits_p[:n_cand], override_candidates, h2[:N_N])
